# Initial kernel scaffold; baseline (speedup 1.0000x reference)
import jax
import jax.numpy as jnp
from jax.experimental import pallas as pl


def kernel(x, edge_index, edge_type, weight_in, w_comp_in, weight_h0, w_comp_h0, weight_out, w_comp_out):
    raise NotImplementedError("write your pallas kernel here")


# speedup vs baseline: 7.6231x; 7.6231x over previous
import functools

import jax
import jax.numpy as jnp
from jax import lax
from jax.experimental import pallas as pl
from jax.experimental.pallas import tpu as pltpu
from jax.experimental.pallas import tpu_sc as plsc

N_NODES = 10000
N_EDGES = 320000
NUM_RELS = 4
NUM_BASES = 2

BLK = 128
N_TILES = 16
E_PAD = 327680
PAD_DST = 10008
N_ACC = 10112
ROWS_PER_TILE = N_ACC // N_TILES
MASK14 = (1 << 14) - 1


def _transform_body(relu_in, x_ref, bases_ref, wcomp_ref, out_ref):
    h = x_ref[...]
    if relu_in:
        h = jnp.maximum(h, 0.0)
    w = wcomp_ref[0, 0, 0] * bases_ref[0] + wcomp_ref[0, 0, 1] * bases_ref[1]
    out_ref[0] = jnp.dot(h, w, preferred_element_type=jnp.float32)


def _transform(h, bases, wcomp, relu_in):
    n, k = h.shape
    d_out = bases.shape[-1]
    nblk = 1000
    grid = (NUM_RELS, n // nblk)
    return pl.pallas_call(
        functools.partial(_transform_body, relu_in),
        grid=grid,
        in_specs=[
            pl.BlockSpec((nblk, k), lambda r, nb: (nb, 0)),
            pl.BlockSpec((NUM_BASES, k, d_out), lambda r, nb: (0, 0, 0)),
            pl.BlockSpec((1, 1, NUM_BASES), lambda r, nb: (r, 0, 0)),
        ],
        out_specs=pl.BlockSpec((1, nblk, d_out), lambda r, nb: (r, nb, 0)),
        out_shape=jax.ShapeDtypeStruct((NUM_RELS, n, d_out), jnp.float32),
    )(h, bases, wcomp.reshape(NUM_RELS, 1, NUM_BASES))


def _relu_body(x_ref, o_ref):
    o_ref[...] = jnp.maximum(x_ref[...], 0.0)


def _relu(x):
    return pl.pallas_call(
        _relu_body,
        out_shape=jax.ShapeDtypeStruct(x.shape, x.dtype),
    )(x)


def _agg_body(dh, nsplit, q0, packed2, table, out, pbuf, idxbuf, dstbuf,
              rows, acc):
    c = lax.axis_index("c")
    s = lax.axis_index("s")
    nblocks = pbuf.shape[0]
    nchunk = dh // 16

    def zero_rows(i, _):
        rows[i // nchunk, pl.ds((i % nchunk) * 16, 16)] = jnp.zeros(
            (16,), jnp.float32)
        return 0

    lax.fori_loop(0, BLK * nchunk, zero_rows, 0)

    base = s * ROWS_PER_TILE
    for rep in range(4):
        pltpu.sync_copy(rows, acc.at[pl.ds(base + rep * BLK, BLK)])
    pltpu.sync_copy(rows.at[pl.ds(0, ROWS_PER_TILE - 4 * BLK)],
                    acc.at[pl.ds(base + 4 * BLK, ROWS_PER_TILE - 4 * BLK)])
    plsc.subcore_barrier()

    roff = s * nblocks
    pltpu.sync_copy(packed2.at[pl.ds(roff, nblocks)], pbuf)

    def calc_idx(i, _):
        j = i // 8
        sl = pl.ds((i % 8) * 16, 16)
        p = pbuf[j, sl]
        dstbuf[j, sl] = p & MASK14
        idxbuf[j, sl] = ((p >> 28) * N_NODES
                         + ((p >> 14) & MASK14)) * nsplit + (q0 + c)
        return 0

    lax.fori_loop(0, nblocks * 8, calc_idx, 0)

    def block(j, _):
        pltpu.sync_copy(table.at[idxbuf.at[j]], rows)
        pltpu.sync_copy(rows, acc.at[dstbuf.at[j]], add=True)
        return 0

    lax.fori_loop(0, nblocks, block, 0)
    plsc.subcore_barrier()

    pltpu.sync_copy(acc.at[pl.ds(base, ROWS_PER_TILE)],
                    out.at[c, pl.ds(base, ROWS_PER_TILE)])


def _aggregate(packed2, table, dh, nsplit, q0):
    nblocks = (E_PAD // BLK) // N_TILES
    mesh = plsc.VectorSubcoreMesh(core_axis_name="c", subcore_axis_name="s")
    return pl.kernel(
        functools.partial(_agg_body, dh, nsplit, q0),
        out_type=jax.ShapeDtypeStruct((2, N_ACC, dh), jnp.float32),
        mesh=mesh,
        scratch_types=[
            pltpu.VMEM((nblocks, BLK), jnp.int32),
            pltpu.VMEM((nblocks, BLK), jnp.int32),
            pltpu.VMEM((nblocks, BLK), jnp.int32),
            pltpu.VMEM((BLK, dh), jnp.float32),
            pltpu.VMEM_SHARED((N_ACC, dh), jnp.float32),
        ],
        compiler_params=pltpu.CompilerParams(use_tc_tiling_on_sc=False),
    )(packed2, table)


def _layer(h, bases, wcomp, relu_in, packed2):
    d_out = bases.shape[-1]
    t = _transform(h, bases, wcomp, relu_in)
    table = t.reshape(NUM_RELS * N_NODES * 4, d_out // 4)
    agg_a = _aggregate(packed2, table, d_out // 4, 4, 0)
    agg_b = _aggregate(packed2, table, d_out // 4, 4, 2)
    return jnp.concatenate(
        [agg_a[0, :N_NODES], agg_a[1, :N_NODES],
         agg_b[0, :N_NODES], agg_b[1, :N_NODES]], axis=1)


def kernel(x, edge_index, edge_type, weight_in, w_comp_in, weight_h0,
           w_comp_h0, weight_out, w_comp_out):
    src = edge_index[0]
    dst = edge_index[1]
    packed = (
        jnp.left_shift(edge_type, 28)
        | jnp.left_shift(src, 14)
        | dst
    )
    pad = E_PAD - N_EDGES
    packed2 = jnp.concatenate(
        [packed, jnp.full((pad,), PAD_DST, jnp.int32)]).reshape(-1, BLK)

    h1 = _layer(x, weight_in, w_comp_in, False, packed2)
    h2 = _layer(h1, weight_h0, w_comp_h0, True, packed2)

    w3 = jnp.broadcast_to(weight_out, (NUM_BASES, weight_out.shape[1], 32))
    t3 = _transform(h2, w3, w_comp_out, True)
    table3 = t3.reshape(NUM_RELS * N_NODES * 2, 16)
    agg3 = _aggregate(packed2, table3, 16, 2, 0)
    out = _relu(agg3[0])
    return out[:N_NODES, 0:1]


# speedup vs baseline: 8.7325x; 1.1455x over previous
import functools

import jax
import jax.numpy as jnp
from jax import lax
from jax.experimental import pallas as pl
from jax.experimental.pallas import tpu as pltpu
from jax.experimental.pallas import tpu_sc as plsc

N_NODES = 10000
N_EDGES = 320000
NUM_RELS = 4
NUM_BASES = 2

BLK = 128
N_TILES = 16
E_PAD = 327680
PAD_DST = 10008
N_ACC = 10112
ROWS_PER_TILE = N_ACC // N_TILES
MASK14 = (1 << 14) - 1


def _transform_body(relu_in, x_ref, bases_ref, wcomp_ref, out_ref):
    h = x_ref[...]
    if relu_in:
        h = jnp.maximum(h, 0.0)
    w = wcomp_ref[0, 0, 0] * bases_ref[0] + wcomp_ref[0, 0, 1] * bases_ref[1]
    out_ref[0] = jnp.dot(h, w, preferred_element_type=jnp.float32)


def _transform(h, bases, wcomp, relu_in):
    n, k = h.shape
    d_out = bases.shape[-1]
    nblk = 1000
    grid = (NUM_RELS, n // nblk)
    return pl.pallas_call(
        functools.partial(_transform_body, relu_in),
        grid=grid,
        in_specs=[
            pl.BlockSpec((nblk, k), lambda r, nb: (nb, 0)),
            pl.BlockSpec((NUM_BASES, k, d_out), lambda r, nb: (0, 0, 0)),
            pl.BlockSpec((1, 1, NUM_BASES), lambda r, nb: (r, 0, 0)),
        ],
        out_specs=pl.BlockSpec((1, nblk, d_out), lambda r, nb: (r, nb, 0)),
        out_shape=jax.ShapeDtypeStruct((NUM_RELS, n, d_out), jnp.float32),
    )(h, bases, wcomp.reshape(NUM_RELS, 1, NUM_BASES))


def _relu_body(x_ref, o_ref):
    o_ref[...] = jnp.maximum(x_ref[...], 0.0)


def _relu(x):
    return pl.pallas_call(
        _relu_body,
        out_shape=jax.ShapeDtypeStruct(x.shape, x.dtype),
    )(x)


NBUF = 4
AHEAD = 2


def _agg_body(dh, nsplit, q0, packed2, table, out, idxbuf, dstbuf, rows,
              acc, gsem, ssem):
    c = lax.axis_index("c")
    s = lax.axis_index("s")
    nblocks = idxbuf.shape[0]
    nchunk = dh // 16

    def zero_rows(i, _):
        rows[0, i // nchunk, pl.ds((i % nchunk) * 16, 16)] = jnp.zeros(
            (16,), jnp.float32)
        return 0

    lax.fori_loop(0, BLK * nchunk, zero_rows, 0)

    base = s * ROWS_PER_TILE
    for rep in range(4):
        pltpu.sync_copy(rows.at[0], acc.at[pl.ds(base + rep * BLK, BLK)])
    pltpu.sync_copy(rows.at[0, pl.ds(0, ROWS_PER_TILE - 4 * BLK)],
                    acc.at[pl.ds(base + 4 * BLK, ROWS_PER_TILE - 4 * BLK)])
    plsc.subcore_barrier()

    roff = s * nblocks
    pltpu.sync_copy(packed2.at[pl.ds(roff, nblocks)], idxbuf)

    def calc_idx(i, _):
        j = i // 8
        sl = pl.ds((i % 8) * 16, 16)
        p = idxbuf[j, sl]
        dstbuf[j, sl] = p & MASK14
        idxbuf[j, sl] = ((p >> 28) * N_NODES
                         + ((p >> 14) & MASK14)) * nsplit + (q0 + c)
        return 0

    lax.fori_loop(0, nblocks * 8, calc_idx, 0)

    def gather(j, b):
        pltpu.async_copy(table.at[idxbuf.at[j]], rows.at[b], gsem)

    def drain_gather():
        pltpu.make_async_copy(table.at[idxbuf.at[0]], rows.at[0], gsem).wait()

    def scatter(j, b):
        pltpu.async_copy(rows.at[b], acc.at[dstbuf.at[j]], ssem, add=True)

    def drain_scatter():
        pltpu.make_async_copy(rows.at[0], acc.at[dstbuf.at[0]], ssem).wait()

    for b in range(AHEAD):
        gather(b, b)

    def outer(o, _):
        for k in range(NBUF):
            j = o * NBUF + k
            drain_gather()
            scatter(j, k)
            jn = j + AHEAD
            nxt = (k + AHEAD) % NBUF

            @pl.when(j >= AHEAD)
            def _():
                drain_scatter()

            @pl.when(jn < nblocks)
            def _():
                gather(jn, nxt)
        return 0

    lax.fori_loop(0, nblocks // NBUF, outer, 0)
    for _ in range(AHEAD):
        drain_scatter()
    plsc.subcore_barrier()

    pltpu.sync_copy(acc.at[pl.ds(base, ROWS_PER_TILE)],
                    out.at[c, pl.ds(base, ROWS_PER_TILE)])


def _aggregate(packed2, table, dh, nsplit, q0):
    nblocks = (E_PAD // BLK) // N_TILES
    mesh = plsc.VectorSubcoreMesh(core_axis_name="c", subcore_axis_name="s")
    return pl.kernel(
        functools.partial(_agg_body, dh, nsplit, q0),
        out_type=jax.ShapeDtypeStruct((2, N_ACC, dh), jnp.float32),
        mesh=mesh,
        scratch_types=[
            pltpu.VMEM((nblocks, BLK), jnp.int32),
            pltpu.VMEM((nblocks, BLK), jnp.int32),
            pltpu.VMEM((NBUF, BLK, dh), jnp.float32),
            pltpu.VMEM_SHARED((N_ACC, dh), jnp.float32),
            pltpu.SemaphoreType.DMA,
            pltpu.SemaphoreType.DMA,
        ],
        compiler_params=pltpu.CompilerParams(use_tc_tiling_on_sc=False),
    )(packed2, table)


def _layer(h, bases, wcomp, relu_in, packed2):
    d_out = bases.shape[-1]
    t = _transform(h, bases, wcomp, relu_in)
    table = t.reshape(NUM_RELS * N_NODES * 4, d_out // 4)
    agg_a = _aggregate(packed2, table, d_out // 4, 4, 0)
    agg_b = _aggregate(packed2, table, d_out // 4, 4, 2)
    return jnp.concatenate(
        [agg_a[0, :N_NODES], agg_a[1, :N_NODES],
         agg_b[0, :N_NODES], agg_b[1, :N_NODES]], axis=1)


def kernel(x, edge_index, edge_type, weight_in, w_comp_in, weight_h0,
           w_comp_h0, weight_out, w_comp_out):
    src = edge_index[0]
    dst = edge_index[1]
    packed = (
        jnp.left_shift(edge_type, 28)
        | jnp.left_shift(src, 14)
        | dst
    )
    pad = E_PAD - N_EDGES
    packed2 = jnp.concatenate(
        [packed, jnp.full((pad,), PAD_DST, jnp.int32)]).reshape(-1, BLK)

    h1 = _layer(x, weight_in, w_comp_in, False, packed2)
    h2 = _layer(h1, weight_h0, w_comp_h0, True, packed2)

    w3 = jnp.broadcast_to(weight_out, (NUM_BASES, weight_out.shape[1], 32))
    t3 = _transform(h2, w3, w_comp_out, True)
    table3 = t3.reshape(NUM_RELS * N_NODES * 2, 16)
    agg3 = _aggregate(packed2, table3, 16, 2, 0)
    out = _relu(agg3[0])
    return out[:N_NODES, 0:1]
